# baseline (speedup 1.0000x reference)
import jax
import jax.numpy as jnp
from jax import lax
from jax.experimental import pallas as pl

_NUM_PILOT = 512


def _find_thresh(w8):
    u = lax.bitcast_convert_type(w8, jnp.uint32)
    top = jnp.uint32(0x80000000)
    key = jnp.where(u >= top, ~u, u | top)

    def body(_, carry):
        p, bit = carry
        cand = p | bit
        cnt = jnp.sum((key >= cand).astype(jnp.int32), keepdims=True).reshape(1, 1)
        return jnp.where(cnt >= _NUM_PILOT, cand, p), lax.shift_right_logical(
            bit, jnp.uint32(1)
        )

    p0 = jnp.zeros((1, 1), jnp.uint32)
    b0 = jnp.full((1, 1), top, jnp.uint32)
    p, _ = lax.fori_loop(0, 32, body, (p0, b0))
    t = jnp.where(p >= top, p ^ top, ~p)
    return lax.bitcast_convert_type(t, jnp.float32)


def _mask_mul_body(x_ref, w_ref, w8_ref, o_ref):
    thresh = _find_thresh(w8_ref[...])
    mask = (w_ref[...] > thresh).astype(jnp.float32)
    o_ref[...] = x_ref[...] * mask


def kernel(inputs, kernel):
    b, d = inputs.shape
    w8 = kernel.reshape(8, d // 8)
    out = pl.pallas_call(
        _mask_mul_body,
        out_shape=jax.ShapeDtypeStruct(inputs.shape, inputs.dtype),
    )(inputs, kernel, w8)
    return out
